# Initial kernel scaffold; baseline (speedup 1.0000x reference)
import jax
import jax.numpy as jnp
from jax.experimental import pallas as pl


def kernel(x, table):
    raise NotImplementedError("write your pallas kernel here")


# speedup vs baseline: 3.0970x; 3.0970x over previous
import jax
import jax.numpy as jnp
from jax.experimental import pallas as pl
from jax.experimental.pallas import tpu as pltpu
from jax.experimental.pallas import tpu_sc as plsc

B = 4096
L = 50
EMB = 128
N = B * L

WINDOW = 128


def _sc_gather(table, idx):
    mesh = plsc.VectorSubcoreMesh(core_axis_name="core",
                                  subcore_axis_name="subcore")

    @pl.kernel(out_type=jax.ShapeDtypeStruct((N, EMB), table.dtype),
               mesh=mesh)
    def gather_kernel(table_hbm, i_hbm, o_hbm):
        def body(i_vmem, o_vmem):
            pltpu.sync_copy(table_hbm.at[i_vmem.at[0]], o_vmem)

        pltpu.emit_pipeline(
            body,
            grid=(N // WINDOW,),
            in_specs=[pl.BlockSpec((1, WINDOW), index_map=lambda i: (0, i))],
            out_specs=[pl.BlockSpec((WINDOW, EMB), index_map=lambda i: (i, 0))],
            core_axis_name=("core", "subcore"),
            dimension_semantics=(pltpu.PARALLEL,),
        )(i_hbm, o_hbm)

    return gather_kernel(table, idx)


def kernel(x, table):
    idx = x.reshape(1, N).astype(jnp.int32)
    out = _sc_gather(table, idx)
    return out.reshape(B, L, EMB)


# speedup vs baseline: 3.2929x; 1.0633x over previous
import jax
import jax.numpy as jnp
from jax.experimental import pallas as pl
from jax.experimental.pallas import tpu as pltpu
from jax.experimental.pallas import tpu_sc as plsc

B = 4096
L = 50
EMB = 128
N = B * L

WINDOW = 256


def _sc_gather(table, idx):
    mesh = plsc.VectorSubcoreMesh(core_axis_name="core",
                                  subcore_axis_name="subcore")

    @pl.kernel(out_type=jax.ShapeDtypeStruct((N, EMB), table.dtype),
               mesh=mesh)
    def gather_kernel(table_hbm, i_hbm, o_hbm):
        def body(i_vmem, o_vmem):
            pltpu.sync_copy(table_hbm.at[i_vmem.at[0]], o_vmem)

        pltpu.emit_pipeline(
            body,
            grid=(N // WINDOW,),
            in_specs=[pl.BlockSpec((1, WINDOW), index_map=lambda i: (0, i))],
            out_specs=[pl.BlockSpec((WINDOW, EMB), index_map=lambda i: (i, 0))],
            core_axis_name=("core", "subcore"),
            dimension_semantics=(pltpu.PARALLEL,),
        )(i_hbm, o_hbm)

    return gather_kernel(table, idx)


def kernel(x, table):
    idx = x.reshape(1, N).astype(jnp.int32)
    out = _sc_gather(table, idx)
    return out.reshape(B, L, EMB)


# speedup vs baseline: 4.2270x; 1.2837x over previous
import jax
import jax.numpy as jnp
from jax.experimental import pallas as pl
from jax.experimental.pallas import tpu as pltpu
from jax.experimental.pallas import tpu_sc as plsc

B = 4096
L = 50
EMB = 128

BLOCK_B = 8


def _sc_gather(table, idx):
    mesh = plsc.VectorSubcoreMesh(core_axis_name="core",
                                  subcore_axis_name="subcore")

    @pl.kernel(out_type=jax.ShapeDtypeStruct((B, L, EMB), table.dtype),
               mesh=mesh)
    def gather_kernel(table_hbm, i_hbm, o_hbm):
        def body(i_vmem, o_vmem):
            for b in range(BLOCK_B):
                pltpu.sync_copy(table_hbm.at[i_vmem.at[b]], o_vmem.at[b])

        pltpu.emit_pipeline(
            body,
            grid=(B // BLOCK_B,),
            in_specs=[pl.BlockSpec((BLOCK_B, L), index_map=lambda i: (i, 0))],
            out_specs=[pl.BlockSpec((BLOCK_B, L, EMB),
                                    index_map=lambda i: (i, 0, 0))],
            core_axis_name=("core", "subcore"),
            dimension_semantics=(pltpu.PARALLEL,),
        )(i_hbm, o_hbm)

    return gather_kernel(table, idx)


def kernel(x, table):
    return _sc_gather(table, x.astype(jnp.int32))


# speedup vs baseline: 5.8794x; 1.3909x over previous
import jax
import jax.numpy as jnp
from jax.experimental import pallas as pl
from jax.experimental.pallas import tpu as pltpu
from jax.experimental.pallas import tpu_sc as plsc

B = 4096
L = 50
EMB = 128

BLOCK_B = 8


def _sc_gather(table, idx):
    mesh = plsc.VectorSubcoreMesh(core_axis_name="core",
                                  subcore_axis_name="subcore")

    @pl.kernel(out_type=jax.ShapeDtypeStruct((B, L, EMB), table.dtype),
               mesh=mesh,
               scratch_types=[pltpu.SemaphoreType.DMA])
    def gather_kernel(table_hbm, i_hbm, o_hbm, sem):
        def body(i_vmem, o_vmem):
            copies = [
                pltpu.make_async_copy(table_hbm.at[i_vmem.at[b]],
                                      o_vmem.at[b], sem)
                for b in range(BLOCK_B)
            ]
            for c in copies:
                c.start()
            for c in copies:
                c.wait()

        pltpu.emit_pipeline(
            body,
            grid=(B // BLOCK_B,),
            in_specs=[pl.BlockSpec((BLOCK_B, L), index_map=lambda i: (i, 0))],
            out_specs=[pl.BlockSpec((BLOCK_B, L, EMB),
                                    index_map=lambda i: (i, 0, 0))],
            core_axis_name=("core", "subcore"),
            dimension_semantics=(pltpu.PARALLEL,),
        )(i_hbm, o_hbm)

    return gather_kernel(table, idx)


def kernel(x, table):
    return _sc_gather(table, x.astype(jnp.int32))
